# baseline (speedup 1.0000x reference)
import jax
import jax.numpy as jnp
from jax.experimental import pallas as pl
from jax.experimental.pallas import tpu as pltpu

_WEIGHT = 1e-7
_SIGMA_RGB = 15.0
_SIGMA_XY = 100.0
_SCALE = 0.5

_BR = 512
_BC = 1024


def _tile_body(f_ref, ft_ref, s_ref, st_ref, hr_ref, hc_ref, o_ref):
    j = pl.program_id(1)
    ff = jax.lax.dot_general(f_ref[0], ft_ref[0], (((1,), (0,)), ((), ())),
                             preferred_element_type=jnp.float32)
    g = jax.lax.dot_general(s_ref[0], st_ref[0], (((1,), (0,)), ((), ())),
                            preferred_element_type=jnp.float32)
    t = jnp.minimum((ff - hr_ref[0]) - hc_ref[0], 0.0)
    part = jnp.sum(jnp.exp(t) * g, axis=0)[None, None, :]

    @pl.when(j == 0)
    def _init():
        o_ref[...] = jnp.zeros_like(o_ref)

    o_ref[...] += part


def kernel(images, segmentations):
    n, k, h, w = segmentations.shape
    img_s = images[:, :, ::2, ::2]
    hs, ws = h // 2, w // 2
    seg_s = segmentations.reshape(n, k, hs, 2, ws, 2).mean(axis=(3, 5))

    P = hs * ws
    yy, xx = jnp.meshgrid(jnp.arange(hs, dtype=jnp.float32),
                          jnp.arange(ws, dtype=jnp.float32), indexing='ij')
    sigma_xy_eff = _SIGMA_XY * _SCALE
    pos = jnp.stack([xx, yy], axis=-1).reshape(P, 2) / sigma_xy_eff
    rgb = img_s.reshape(n, 3, P).transpose(0, 2, 1) / _SIGMA_RGB
    feats = jnp.concatenate(
        [jnp.broadcast_to(pos[None], (n, P, 2)), rgb], axis=-1)
    sq = jnp.sum(feats * feats, axis=-1)
    half = 0.5 * sq

    F = jnp.concatenate(
        [feats, jnp.zeros((n, P, 3), jnp.float32)], axis=-1)
    FT = F.transpose(0, 2, 1)
    ST = jnp.concatenate(
        [seg_s.reshape(n, k, P),
         jnp.zeros((n, 8 - k, P), jnp.float32)], axis=1)
    S = ST.transpose(0, 2, 1)
    hr = half[:, :, None]
    hc = half[:, None, :]

    rb = P // _BR
    cb = P // _BC
    g0 = n * rb

    partials = pl.pallas_call(
        _tile_body,
        out_shape=jax.ShapeDtypeStruct((g0, 1, _BC), jnp.float32),
        grid=(g0, cb),
        in_specs=[
            pl.BlockSpec((1, _BR, 8), lambda i, j: (i // rb, i % rb, 0)),
            pl.BlockSpec((1, 8, _BC), lambda i, j: (i // rb, 0, j)),
            pl.BlockSpec((1, _BR, 8), lambda i, j: (i // rb, i % rb, 0)),
            pl.BlockSpec((1, 8, _BC), lambda i, j: (i // rb, 0, j)),
            pl.BlockSpec((1, _BR, 1), lambda i, j: (i // rb, i % rb, 0)),
            pl.BlockSpec((1, 1, _BC), lambda i, j: (i // rb, 0, j)),
        ],
        out_specs=pl.BlockSpec((1, 1, _BC), lambda i, j: (i, 0, 0)),
        compiler_params=pltpu.CompilerParams(
            dimension_semantics=("parallel", "arbitrary"),
        ),
        name="dense_crf_loss",
    )(F, FT, S, ST, hr, hc)

    loss = -jnp.sum(partials) / jnp.float32(n)
    return (_WEIGHT * loss).reshape(1)


# speedup vs baseline: 1.0205x; 1.0205x over previous
import jax
import jax.numpy as jnp
from jax.experimental import pallas as pl
from jax.experimental.pallas import tpu as pltpu

_WEIGHT = 1e-7
_SIGMA_RGB = 15.0
_SIGMA_XY = 100.0
_SCALE = 0.5

_BR = 512
_BC = 1024


def _tile_body(ftr_ref, ftc_ref, str_ref, stc_ref, hr_ref, hc_ref, o_ref):
    j = pl.program_id(1)
    ff = jax.lax.dot_general(ftr_ref[0], ftc_ref[0], (((0,), (0,)), ((), ())),
                             preferred_element_type=jnp.float32)
    g = jax.lax.dot_general(str_ref[0], stc_ref[0], (((0,), (0,)), ((), ())),
                            preferred_element_type=jnp.float32)
    t = jnp.minimum((ff - hr_ref[0]) - hc_ref[0], 0.0)
    part = jnp.sum(jnp.exp(t) * g, axis=0)[None, None, :]

    @pl.when(j == 0)
    def _init():
        o_ref[...] = jnp.zeros_like(o_ref)

    o_ref[...] += part


def kernel(images, segmentations):
    n, k, h, w = segmentations.shape
    img_s = images[:, :, ::2, ::2]
    hs, ws = h // 2, w // 2
    seg_s = segmentations.reshape(n, k, hs, 2, ws, 2).mean(axis=(3, 5))

    P = hs * ws
    yy, xx = jnp.meshgrid(jnp.arange(hs, dtype=jnp.float32),
                          jnp.arange(ws, dtype=jnp.float32), indexing='ij')
    sigma_xy_eff = _SIGMA_XY * _SCALE
    pos_rows = jnp.stack([xx, yy], axis=0).reshape(2, P) / sigma_xy_eff
    rgb_rows = img_s.reshape(n, 3, P) / _SIGMA_RGB
    FT = jnp.concatenate(
        [jnp.broadcast_to(pos_rows[None], (n, 2, P)), rgb_rows,
         jnp.zeros((n, 3, P), jnp.float32)], axis=1)
    ST = jnp.concatenate(
        [seg_s.reshape(n, k, P),
         jnp.zeros((n, 8 - k, P), jnp.float32)], axis=1)
    half = 0.5 * jnp.sum(FT * FT, axis=1)
    hr = half[:, :, None]
    hc = half[:, None, :]

    rb = P // _BR
    cb = P // _BC
    g0 = n * rb

    partials = pl.pallas_call(
        _tile_body,
        out_shape=jax.ShapeDtypeStruct((g0, 1, _BC), jnp.float32),
        grid=(g0, cb),
        in_specs=[
            pl.BlockSpec((1, 8, _BR), lambda i, j: (i // rb, 0, i % rb)),
            pl.BlockSpec((1, 8, _BC), lambda i, j: (i // rb, 0, j)),
            pl.BlockSpec((1, 8, _BR), lambda i, j: (i // rb, 0, i % rb)),
            pl.BlockSpec((1, 8, _BC), lambda i, j: (i // rb, 0, j)),
            pl.BlockSpec((1, _BR, 1), lambda i, j: (i // rb, i % rb, 0)),
            pl.BlockSpec((1, 1, _BC), lambda i, j: (i // rb, 0, j)),
        ],
        out_specs=pl.BlockSpec((1, 1, _BC), lambda i, j: (i, 0, 0)),
        compiler_params=pltpu.CompilerParams(
            dimension_semantics=("parallel", "arbitrary"),
        ),
        name="dense_crf_loss",
    )(FT, FT, ST, ST, hr, hc)

    loss = -jnp.sum(partials) / jnp.float32(n)
    return (_WEIGHT * loss).reshape(1)


# speedup vs baseline: 1.5108x; 1.4805x over previous
import jax
import jax.numpy as jnp
from jax.experimental import pallas as pl
from jax.experimental.pallas import tpu as pltpu

_WEIGHT = 1e-7
_SIGMA_RGB = 15.0
_SIGMA_XY = 100.0
_SCALE = 0.5

_BR = 512
_BC = 1024


def _prep_body(img_ref, seg_ref, ft_ref, st_ref, hc_ref):
    hs = img_ref.shape[2] // 2
    ws = img_ref.shape[3] // 2
    p_total = hs * ws
    wfull = img_ref.shape[3]
    v4 = img_ref[0].reshape(3, hs, 2, wfull)
    vr = v4[:, :, 0, :]
    vt = jnp.transpose(vr, (0, 2, 1))
    vt4 = vt.reshape(3, ws, 2, hs)
    img_xy = vt4[:, :, 0, :] / _SIGMA_RGB
    kk = seg_ref.shape[1]
    s4 = seg_ref[0].reshape(kk, hs, 2, wfull)
    sr = s4[:, :, 0, :] + s4[:, :, 1, :]
    st = jnp.transpose(sr, (0, 2, 1))
    st4 = st.reshape(kk, ws, 2, hs)
    seg_xy = (st4[:, :, 0, :] + st4[:, :, 1, :]) * 0.25

    k = seg_xy.shape[0]
    for x in range(ws):
        sl = slice(hs * x, hs * (x + 1))
        ft_ref[0, 2:5, sl] = img_xy[:, x, :]
        st_ref[0, 0:k, sl] = seg_xy[:, x, :]

    lane = jax.lax.broadcasted_iota(jnp.int32, (1, p_total), 1)
    sigma_xy_eff = _SIGMA_XY * _SCALE
    ft_ref[0, 0:1, :] = (lane // hs).astype(jnp.float32) / sigma_xy_eff
    ft_ref[0, 1:2, :] = (lane % hs).astype(jnp.float32) / sigma_xy_eff
    ft_ref[0, 5:8, :] = jnp.zeros((3, p_total), jnp.float32)
    st_ref[0, k:8, :] = jnp.zeros((8 - k, p_total), jnp.float32)
    f = ft_ref[0]
    hc_ref[0] = 0.5 * jnp.sum(f * f, axis=0, keepdims=True)


def _tile_body(ftr_ref, ftc_ref, str_ref, stc_ref, hc_ref, o_ref, hr_ref):
    j = pl.program_id(1)

    @pl.when(j == 0)
    def _make_hr():
        frt = jnp.transpose(ftr_ref[0])
        hr_ref[...] = 0.5 * jnp.sum(frt * frt, axis=1, keepdims=True)

    ff = jax.lax.dot_general(ftr_ref[0], ftc_ref[0], (((0,), (0,)), ((), ())),
                             preferred_element_type=jnp.float32)
    g = jax.lax.dot_general(str_ref[0], stc_ref[0], (((0,), (0,)), ((), ())),
                            preferred_element_type=jnp.float32)
    t = jnp.minimum((ff - hr_ref[...]) - hc_ref[0], 0.0)
    part = jnp.sum(jnp.exp(t) * g, axis=0)[None, None, :]

    @pl.when(j == 0)
    def _init():
        o_ref[...] = jnp.zeros_like(o_ref)

    o_ref[...] += part


def kernel(images, segmentations):
    n, k, h, w = segmentations.shape
    hs, ws = h // 2, w // 2
    P = hs * ws

    FT, ST, HC = pl.pallas_call(
        _prep_body,
        out_shape=[
            jax.ShapeDtypeStruct((n, 8, P), jnp.float32),
            jax.ShapeDtypeStruct((n, 8, P), jnp.float32),
            jax.ShapeDtypeStruct((n, 1, P), jnp.float32),
        ],
        grid=(n,),
        in_specs=[
            pl.BlockSpec((1, 3, h, w), lambda b: (b, 0, 0, 0)),
            pl.BlockSpec((1, k, h, w), lambda b: (b, 0, 0, 0)),
        ],
        out_specs=[
            pl.BlockSpec((1, 8, P), lambda b: (b, 0, 0)),
            pl.BlockSpec((1, 8, P), lambda b: (b, 0, 0)),
            pl.BlockSpec((1, 1, P), lambda b: (b, 0, 0)),
        ],
        compiler_params=pltpu.CompilerParams(
            dimension_semantics=("arbitrary",),
        ),
        name="dense_crf_prep",
    )(images, segmentations)

    rb = P // _BR
    cb = P // _BC
    g0 = n * rb

    partials = pl.pallas_call(
        _tile_body,
        out_shape=jax.ShapeDtypeStruct((g0, 1, _BC), jnp.float32),
        grid=(g0, cb),
        in_specs=[
            pl.BlockSpec((1, 8, _BR), lambda i, j: (i // rb, 0, i % rb)),
            pl.BlockSpec((1, 8, _BC), lambda i, j: (i // rb, 0, j)),
            pl.BlockSpec((1, 8, _BR), lambda i, j: (i // rb, 0, i % rb)),
            pl.BlockSpec((1, 8, _BC), lambda i, j: (i // rb, 0, j)),
            pl.BlockSpec((1, 1, _BC), lambda i, j: (i // rb, 0, j)),
        ],
        out_specs=pl.BlockSpec((1, 1, _BC), lambda i, j: (i, 0, 0)),
        scratch_shapes=[pltpu.VMEM((_BR, 1), jnp.float32)],
        compiler_params=pltpu.CompilerParams(
            dimension_semantics=("parallel", "arbitrary"),
        ),
        name="dense_crf_loss",
    )(FT, FT, ST, ST, HC)

    loss = -jnp.sum(partials) / jnp.float32(n)
    return (_WEIGHT * loss).reshape(1)
